# baseline (speedup 1.0000x reference)
import jax
import jax.numpy as jnp
from jax.experimental import pallas as pl
from jax.experimental.pallas import tpu as pltpu
from jax.experimental.pallas import tpu_sc as plsc

B, N, D = 16, 576, 256
K = 8192
T = B * N
M_TILE = 256
N_TILES = T // M_TILE
GATHER_WINDOW = 128
GATHER_STEPS = T // GATHER_WINDOW


def _c2_body(cb_ref, c2_ref, cb2_ref):
    cb = cb_ref[...]
    c2_ref[...] = jnp.sum(cb * cb, axis=1).reshape(1, K)
    cb2_ref[...] = (cb + cb).astype(jnp.bfloat16)


LANE = 128
N_LANE_BLOCKS = K // LANE


def _argmin_body(x_ref, cb2_ref, c2_ref, idx_ref):
    x = x_ref[...]
    xc2 = jax.lax.dot_general(
        x.astype(jnp.bfloat16), cb2_ref[...], (((1,), (1,)), ((), ())),
        preferred_element_type=jnp.float32)
    x2 = jnp.sum(x * x, axis=1, keepdims=True)
    c2 = c2_ref[...]

    def block_dist(j):
        return (x2 - xc2[:, j * LANE:(j + 1) * LANE]) \
            + c2[:, j * LANE:(j + 1) * LANE]

    acc_v = block_dist(0)
    acc_b = jnp.zeros((M_TILE, LANE), jnp.int32)
    for j in range(1, N_LANE_BLOCKS):
        d = block_dist(j)
        cmp = d < acc_v
        acc_v = jnp.minimum(acc_v, d)
        acc_b = jnp.where(cmp, jnp.int32(j), acc_b)

    mn = jnp.min(acc_v, axis=1, keepdims=True)
    lane = jax.lax.broadcasted_iota(jnp.int32, (M_TILE, LANE), 1)
    kk = acc_b * LANE + lane
    idx = jnp.min(jnp.where(acc_v == mn, kk, jnp.int32(K)), axis=1)
    idx_ref[0, 0, :] = idx


def _codebook_prep(codebook):
    return pl.pallas_call(
        _c2_body,
        in_specs=[pl.BlockSpec((K, D), lambda: (0, 0))],
        out_specs=[pl.BlockSpec((1, K), lambda: (0, 0)),
                   pl.BlockSpec((K, D), lambda: (0, 0))],
        out_shape=[jax.ShapeDtypeStruct((1, K), jnp.float32),
                   jax.ShapeDtypeStruct((K, D), jnp.bfloat16)],
    )(codebook)


def _nearest_idx(x_flat, cb2, c2, tile0, tiles):
    out = pl.pallas_call(
        _argmin_body,
        grid=(tiles,),
        in_specs=[
            pl.BlockSpec((M_TILE, D), lambda i: (i + tile0, 0)),
            pl.BlockSpec((K, D), lambda i: (0, 0)),
            pl.BlockSpec((1, K), lambda i: (0, 0)),
        ],
        out_specs=pl.BlockSpec((1, 1, M_TILE), lambda i: (i, 0, 0)),
        out_shape=jax.ShapeDtypeStruct((tiles, 1, M_TILE), jnp.int32),
        compiler_params=pltpu.CompilerParams(
            dimension_semantics=("parallel",)),
    )(x_flat, cb2, c2)
    return out.reshape(tiles * M_TILE)


def _sc_mesh():
    return plsc.VectorSubcoreMesh(
        core_axis_name="core", subcore_axis_name="subcore")


def _sc_gather(codebook, idx):
    n = idx.shape[0]
    idx2 = idx.reshape(1, n)

    @pl.kernel(out_type=jax.ShapeDtypeStruct((n, D), codebook.dtype),
               mesh=_sc_mesh())
    def kern(cb_hbm, i_hbm, o_hbm):
        def body(i_vmem, o_vmem):
            pltpu.sync_copy(cb_hbm.at[i_vmem.at[0]], o_vmem)

        pltpu.emit_pipeline(
            body,
            grid=(n // GATHER_WINDOW,),
            in_specs=[pl.BlockSpec((1, GATHER_WINDOW),
                                   index_map=lambda i: (0, i))],
            out_specs=[pl.BlockSpec((GATHER_WINDOW, D),
                                    index_map=lambda i: (i, 0))],
            core_axis_name=("core", "subcore"),
            dimension_semantics=(pltpu.PARALLEL,),
        )(i_hbm, o_hbm)

    return kern(codebook, idx2)


def _sc_gather_combine(codebook, idx, part0):
    n = idx.shape[0]
    h = part0.shape[0]
    idx2 = idx.reshape(1, n)
    win0 = h // GATHER_WINDOW
    rows_per_unit = h // 32

    @pl.kernel(out_type=jax.ShapeDtypeStruct((T, D), codebook.dtype),
               mesh=_sc_mesh())
    def kern(cb_hbm, i_hbm, p0_hbm, o_hbm):
        u = jax.lax.axis_index("core") * 16 + jax.lax.axis_index("subcore")
        r0 = u * rows_per_unit
        pltpu.sync_copy(p0_hbm.at[pl.ds(r0, rows_per_unit)],
                        o_hbm.at[pl.ds(r0, rows_per_unit)])

        def body(i_vmem, o_vmem):
            pltpu.sync_copy(cb_hbm.at[i_vmem.at[0]], o_vmem)

        pltpu.emit_pipeline(
            body,
            grid=(n // GATHER_WINDOW,),
            in_specs=[pl.BlockSpec((1, GATHER_WINDOW),
                                   index_map=lambda i: (0, i))],
            out_specs=[pl.BlockSpec((GATHER_WINDOW, D),
                                    index_map=lambda i: (i + win0, 0))],
            core_axis_name=("core", "subcore"),
            dimension_semantics=(pltpu.PARALLEL,),
        )(i_hbm, o_hbm)

    return kern(codebook, idx2, part0)


N_CHUNKS = 2


def kernel(uncompressed, mask, codebook):
    x_flat = uncompressed.reshape(T, D)
    c2, cb2 = _codebook_prep(codebook)
    tiles_per_chunk = N_TILES // N_CHUNKS
    idx0 = _nearest_idx(x_flat, cb2, c2, 0, tiles_per_chunk)
    part0 = _sc_gather(codebook, idx0)
    idx1 = _nearest_idx(x_flat, cb2, c2, tiles_per_chunk, tiles_per_chunk)
    compressed = _sc_gather_combine(codebook, idx1, part0).reshape(B, N, D)
    return (compressed, uncompressed, mask, codebook)


# speedup vs baseline: 1.9303x; 1.9303x over previous
import jax
import jax.numpy as jnp
from jax.experimental import pallas as pl
from jax.experimental.pallas import tpu as pltpu
from jax.experimental.pallas import tpu_sc as plsc

B, N, D = 16, 576, 256
K = 8192
T = B * N
M_TILE = 256
N_TILES = T // M_TILE
GATHER_WINDOW = 128
GATHER_STEPS = T // GATHER_WINDOW


def _c2_body(cb_ref, c2_ref, cb2_ref):
    cb = cb_ref[...]
    c2_ref[...] = jnp.sum(cb * cb, axis=1).reshape(1, K)
    cb2_ref[...] = (cb + cb).astype(jnp.bfloat16)


LANE = 128
N_LANE_BLOCKS = K // LANE


def _argmin_body(x_ref, cb2_ref, c2_ref, idx_ref):
    x = x_ref[...]
    xc2 = jax.lax.dot_general(
        x.astype(jnp.bfloat16), cb2_ref[...], (((1,), (1,)), ((), ())),
        preferred_element_type=jnp.float32)
    x2 = jnp.sum(x * x, axis=1, keepdims=True)
    c2 = c2_ref[...]

    def block_dist(j):
        return (x2 - xc2[:, j * LANE:(j + 1) * LANE]) \
            + c2[:, j * LANE:(j + 1) * LANE]

    acc_v = block_dist(0)
    acc_b = jnp.zeros((M_TILE, LANE), jnp.int32)
    for j in range(1, N_LANE_BLOCKS):
        d = block_dist(j)
        cmp = d < acc_v
        acc_v = jnp.minimum(acc_v, d)
        acc_b = jnp.where(cmp, jnp.int32(j), acc_b)

    mn = jnp.min(acc_v, axis=1, keepdims=True)
    lane = jax.lax.broadcasted_iota(jnp.int32, (M_TILE, LANE), 1)
    kk = acc_b * LANE + lane
    idx = jnp.min(jnp.where(acc_v == mn, kk, jnp.int32(K)), axis=1)
    idx_ref[0, 0, :] = idx


def _codebook_prep(codebook):
    return pl.pallas_call(
        _c2_body,
        in_specs=[pl.BlockSpec((K, D), lambda: (0, 0))],
        out_specs=[pl.BlockSpec((1, K), lambda: (0, 0)),
                   pl.BlockSpec((K, D), lambda: (0, 0))],
        out_shape=[jax.ShapeDtypeStruct((1, K), jnp.float32),
                   jax.ShapeDtypeStruct((K, D), jnp.bfloat16)],
    )(codebook)


def _nearest_idx(x_flat, cb2, c2, tile0, tiles):
    out = pl.pallas_call(
        _argmin_body,
        grid=(tiles,),
        in_specs=[
            pl.BlockSpec((M_TILE, D), lambda i: (i + tile0, 0)),
            pl.BlockSpec((K, D), lambda i: (0, 0)),
            pl.BlockSpec((1, K), lambda i: (0, 0)),
        ],
        out_specs=pl.BlockSpec((1, 1, M_TILE), lambda i: (i, 0, 0)),
        out_shape=jax.ShapeDtypeStruct((tiles, 1, M_TILE), jnp.int32),
        compiler_params=pltpu.CompilerParams(
            dimension_semantics=("parallel",)),
    )(x_flat, cb2, c2)
    return out.reshape(tiles * M_TILE)


def _sc_mesh():
    return plsc.VectorSubcoreMesh(
        core_axis_name="core", subcore_axis_name="subcore")


def _sc_gather(codebook, idx):
    n = idx.shape[0]
    idx2 = idx.reshape(1, n)
    win = GATHER_WINDOW

    @pl.kernel(out_type=jax.ShapeDtypeStruct((n, D), codebook.dtype),
               mesh=_sc_mesh())
    def kern(cb_hbm, i_hbm, o_hbm):
        def body(i_vmem, o_vmem):
            pltpu.sync_copy(cb_hbm.at[i_vmem.at[0]], o_vmem)

        pltpu.emit_pipeline(
            body,
            grid=(n // win,),
            in_specs=[pl.BlockSpec((1, win),
                                   index_map=lambda i: (0, i))],
            out_specs=[pl.BlockSpec((win, D),
                                    index_map=lambda i: (i, 0))],
            core_axis_name=("core", "subcore"),
            dimension_semantics=(pltpu.PARALLEL,),
        )(i_hbm, o_hbm)

    return kern(codebook, idx2)


def _sc_gather_combine(codebook, idx, part0):
    n = idx.shape[0]
    h = part0.shape[0]
    idx2 = idx.reshape(1, n)
    win0 = h // GATHER_WINDOW
    rows_per_unit = h // 32

    @pl.kernel(out_type=jax.ShapeDtypeStruct((T, D), codebook.dtype),
               mesh=_sc_mesh())
    def kern(cb_hbm, i_hbm, p0_hbm, o_hbm):
        u = jax.lax.axis_index("core") * 16 + jax.lax.axis_index("subcore")
        r0 = u * rows_per_unit
        pltpu.sync_copy(p0_hbm.at[pl.ds(r0, rows_per_unit)],
                        o_hbm.at[pl.ds(r0, rows_per_unit)])

        def body(i_vmem, o_vmem):
            pltpu.sync_copy(cb_hbm.at[i_vmem.at[0]], o_vmem)

        pltpu.emit_pipeline(
            body,
            grid=(n // GATHER_WINDOW,),
            in_specs=[pl.BlockSpec((1, GATHER_WINDOW),
                                   index_map=lambda i: (0, i))],
            out_specs=[pl.BlockSpec((GATHER_WINDOW, D),
                                    index_map=lambda i: (i + win0, 0))],
            core_axis_name=("core", "subcore"),
            dimension_semantics=(pltpu.PARALLEL,),
        )(i_hbm, o_hbm)

    return kern(codebook, idx2, part0)


CHUNK_TILES = (24, 12)


def kernel(uncompressed, mask, codebook):
    x_flat = uncompressed.reshape(T, D)
    c2, cb2 = _codebook_prep(codebook)
    parts = []
    tile0 = 0
    for tiles in CHUNK_TILES:
        idx_c = _nearest_idx(x_flat, cb2, c2, tile0, tiles)
        parts.append(_sc_gather(codebook, idx_c))
        tile0 += tiles
    compressed = jnp.concatenate(parts, axis=0).reshape(B, N, D)
    return (compressed, uncompressed, mask, codebook)


# speedup vs baseline: 1.9449x; 1.0076x over previous
import jax
import jax.numpy as jnp
from jax.experimental import pallas as pl
from jax.experimental.pallas import tpu as pltpu
from jax.experimental.pallas import tpu_sc as plsc

B, N, D = 16, 576, 256
K = 8192
T = B * N
M_TILE = 256
N_TILES = T // M_TILE
GATHER_WINDOW = 128
GATHER_STEPS = T // GATHER_WINDOW


def _c2_body(cb_ref, c2_ref, cb2_ref):
    cb = cb_ref[...]
    c2_ref[...] = jnp.sum(cb * cb, axis=1).reshape(1, K)
    cb2_ref[...] = (cb + cb).astype(jnp.bfloat16)


LANE = 128
N_LANE_BLOCKS = K // LANE


def _argmin_body(x_ref, cb2_ref, c2_ref, idx_ref):
    x = x_ref[...]
    xc2 = jax.lax.dot_general(
        x.astype(jnp.bfloat16), cb2_ref[...], (((1,), (1,)), ((), ())),
        preferred_element_type=jnp.float32)
    x2 = jnp.sum(x * x, axis=1, keepdims=True)
    c2 = c2_ref[...]

    def block_dist(j):
        return (x2 - xc2[:, j * LANE:(j + 1) * LANE]) \
            + c2[:, j * LANE:(j + 1) * LANE]

    acc_v = block_dist(0)
    acc_b = jnp.zeros((M_TILE, LANE), jnp.int32)
    for j in range(1, N_LANE_BLOCKS):
        d = block_dist(j)
        cmp = d < acc_v
        acc_v = jnp.minimum(acc_v, d)
        acc_b = jnp.where(cmp, jnp.int32(j), acc_b)

    mn = jnp.min(acc_v, axis=1, keepdims=True)
    lane = jax.lax.broadcasted_iota(jnp.int32, (M_TILE, LANE), 1)
    kk = acc_b * LANE + lane
    idx = jnp.min(jnp.where(acc_v == mn, kk, jnp.int32(K)), axis=1)
    idx_ref[0, 0, :] = idx


def _codebook_prep(codebook):
    return pl.pallas_call(
        _c2_body,
        in_specs=[pl.BlockSpec((K, D), lambda: (0, 0))],
        out_specs=[pl.BlockSpec((1, K), lambda: (0, 0)),
                   pl.BlockSpec((K, D), lambda: (0, 0))],
        out_shape=[jax.ShapeDtypeStruct((1, K), jnp.float32),
                   jax.ShapeDtypeStruct((K, D), jnp.bfloat16)],
    )(codebook)


def _nearest_idx(x_flat, cb2, c2, tile0, tiles):
    out = pl.pallas_call(
        _argmin_body,
        grid=(tiles,),
        in_specs=[
            pl.BlockSpec((M_TILE, D), lambda i: (i + tile0, 0)),
            pl.BlockSpec((K, D), lambda i: (0, 0)),
            pl.BlockSpec((1, K), lambda i: (0, 0)),
        ],
        out_specs=pl.BlockSpec((1, 1, M_TILE), lambda i: (i, 0, 0)),
        out_shape=jax.ShapeDtypeStruct((tiles, 1, M_TILE), jnp.int32),
        compiler_params=pltpu.CompilerParams(
            dimension_semantics=("parallel",)),
    )(x_flat, cb2, c2)
    return out.reshape(tiles * M_TILE)


def _sc_mesh():
    return plsc.VectorSubcoreMesh(
        core_axis_name="core", subcore_axis_name="subcore")


def _sc_gather(codebook, idx):
    n = idx.shape[0]
    idx2 = idx.reshape(1, n)
    win = GATHER_WINDOW

    @pl.kernel(out_type=jax.ShapeDtypeStruct((n, D), codebook.dtype),
               mesh=_sc_mesh())
    def kern(cb_hbm, i_hbm, o_hbm):
        def body(i_vmem, o_vmem):
            pltpu.sync_copy(cb_hbm.at[i_vmem.at[0]], o_vmem)

        pltpu.emit_pipeline(
            body,
            grid=(n // win,),
            in_specs=[pl.BlockSpec((1, win),
                                   index_map=lambda i: (0, i))],
            out_specs=[pl.BlockSpec((win, D),
                                    index_map=lambda i: (i, 0))],
            core_axis_name=("core", "subcore"),
            dimension_semantics=(pltpu.PARALLEL,),
        )(i_hbm, o_hbm)

    return kern(codebook, idx2)


def _sc_gather_combine(codebook, idx, part0):
    n = idx.shape[0]
    h = part0.shape[0]
    idx2 = idx.reshape(1, n)
    win0 = h // GATHER_WINDOW
    rows_per_unit = h // 32

    @pl.kernel(out_type=jax.ShapeDtypeStruct((T, D), codebook.dtype),
               mesh=_sc_mesh())
    def kern(cb_hbm, i_hbm, p0_hbm, o_hbm):
        u = jax.lax.axis_index("core") * 16 + jax.lax.axis_index("subcore")
        r0 = u * rows_per_unit
        pltpu.sync_copy(p0_hbm.at[pl.ds(r0, rows_per_unit)],
                        o_hbm.at[pl.ds(r0, rows_per_unit)])

        def body(i_vmem, o_vmem):
            pltpu.sync_copy(cb_hbm.at[i_vmem.at[0]], o_vmem)

        pltpu.emit_pipeline(
            body,
            grid=(n // GATHER_WINDOW,),
            in_specs=[pl.BlockSpec((1, GATHER_WINDOW),
                                   index_map=lambda i: (0, i))],
            out_specs=[pl.BlockSpec((GATHER_WINDOW, D),
                                    index_map=lambda i: (i + win0, 0))],
            core_axis_name=("core", "subcore"),
            dimension_semantics=(pltpu.PARALLEL,),
        )(i_hbm, o_hbm)

    return kern(codebook, idx2, part0)


CHUNK_TILES = (18, 18)


def kernel(uncompressed, mask, codebook):
    x_flat = uncompressed.reshape(T, D)
    c2, cb2 = _codebook_prep(codebook)
    parts = []
    tile0 = 0
    for tiles in CHUNK_TILES:
        idx_c = _nearest_idx(x_flat, cb2, c2, tile0, tiles)
        parts.append(_sc_gather(codebook, idx_c))
        tile0 += tiles
    compressed = jnp.concatenate(parts, axis=0).reshape(B, N, D)
    return (compressed, uncompressed, mask, codebook)


# speedup vs baseline: 1.9452x; 1.0002x over previous
import jax
import jax.numpy as jnp
from jax.experimental import pallas as pl
from jax.experimental.pallas import tpu as pltpu
from jax.experimental.pallas import tpu_sc as plsc

B, N, D = 16, 576, 256
K = 8192
T = B * N
M_TILE = 256
N_TILES = T // M_TILE
GATHER_WINDOW = 128
GATHER_STEPS = T // GATHER_WINDOW


def _c2_body(cb_ref, c2_ref, cb2_ref):
    cb = cb_ref[...]
    c2_ref[...] = jnp.sum(cb * cb, axis=1).reshape(1, K)
    cb2_ref[...] = (cb + cb).astype(jnp.bfloat16)


LANE = 128
N_LANE_BLOCKS = K // LANE


K_SPLIT = 4
KS = K // K_SPLIT


def _argmin_body(x_ref, cb2_ref, c2_ref, idx_ref):
    x = x_ref[...]
    x16 = x.astype(jnp.bfloat16)
    x2 = jnp.sum(x * x, axis=1, keepdims=True)
    c2 = c2_ref[...]
    lane = jax.lax.broadcasted_iota(jnp.int32, (M_TILE, LANE), 1)

    acc_v = None
    for s in range(K_SPLIT):
        xc2 = jax.lax.dot_general(
            x16, cb2_ref[s * KS:(s + 1) * KS, :], (((1,), (1,)), ((), ())),
            preferred_element_type=jnp.float32)

        def block_dist(j):
            return (x2 - xc2[:, j * LANE:(j + 1) * LANE]) \
                + c2[:, (s * KS + j * LANE):(s * KS + (j + 1) * LANE)]

        j0 = s * (KS // LANE)
        for j in range(KS // LANE):
            d = block_dist(j)
            if acc_v is None:
                acc_v = d
                acc_b = jnp.zeros((M_TILE, LANE), jnp.int32)
            else:
                cmp = d < acc_v
                acc_v = jnp.minimum(acc_v, d)
                acc_b = jnp.where(cmp, jnp.int32(j0 + j), acc_b)

    mn = jnp.min(acc_v, axis=1, keepdims=True)
    kk = acc_b * LANE + lane
    idx = jnp.min(jnp.where(acc_v == mn, kk, jnp.int32(K)), axis=1)
    idx_ref[0, 0, :] = idx


def _codebook_prep(codebook):
    return pl.pallas_call(
        _c2_body,
        in_specs=[pl.BlockSpec((K, D), lambda: (0, 0))],
        out_specs=[pl.BlockSpec((1, K), lambda: (0, 0)),
                   pl.BlockSpec((K, D), lambda: (0, 0))],
        out_shape=[jax.ShapeDtypeStruct((1, K), jnp.float32),
                   jax.ShapeDtypeStruct((K, D), jnp.bfloat16)],
    )(codebook)


def _nearest_idx(x_flat, cb2, c2, tile0, tiles):
    out = pl.pallas_call(
        _argmin_body,
        grid=(tiles,),
        in_specs=[
            pl.BlockSpec((M_TILE, D), lambda i: (i + tile0, 0)),
            pl.BlockSpec((K, D), lambda i: (0, 0)),
            pl.BlockSpec((1, K), lambda i: (0, 0)),
        ],
        out_specs=pl.BlockSpec((1, 1, M_TILE), lambda i: (i, 0, 0)),
        out_shape=jax.ShapeDtypeStruct((tiles, 1, M_TILE), jnp.int32),
        compiler_params=pltpu.CompilerParams(
            dimension_semantics=("parallel",)),
    )(x_flat, cb2, c2)
    return out.reshape(tiles * M_TILE)


def _sc_mesh():
    return plsc.VectorSubcoreMesh(
        core_axis_name="core", subcore_axis_name="subcore")


def _sc_gather(codebook, idx):
    n = idx.shape[0]
    idx2 = idx.reshape(1, n)
    win = GATHER_WINDOW

    @pl.kernel(out_type=jax.ShapeDtypeStruct((n, D), codebook.dtype),
               mesh=_sc_mesh())
    def kern(cb_hbm, i_hbm, o_hbm):
        def body(i_vmem, o_vmem):
            pltpu.sync_copy(cb_hbm.at[i_vmem.at[0]], o_vmem)

        pltpu.emit_pipeline(
            body,
            grid=(n // win,),
            in_specs=[pl.BlockSpec((1, win),
                                   index_map=lambda i: (0, i))],
            out_specs=[pl.BlockSpec((win, D),
                                    index_map=lambda i: (i, 0))],
            core_axis_name=("core", "subcore"),
            dimension_semantics=(pltpu.PARALLEL,),
        )(i_hbm, o_hbm)

    return kern(codebook, idx2)


def _sc_gather_combine(codebook, idx, part0):
    n = idx.shape[0]
    h = part0.shape[0]
    idx2 = idx.reshape(1, n)
    win0 = h // GATHER_WINDOW
    rows_per_unit = h // 32

    @pl.kernel(out_type=jax.ShapeDtypeStruct((T, D), codebook.dtype),
               mesh=_sc_mesh())
    def kern(cb_hbm, i_hbm, p0_hbm, o_hbm):
        u = jax.lax.axis_index("core") * 16 + jax.lax.axis_index("subcore")
        r0 = u * rows_per_unit
        pltpu.sync_copy(p0_hbm.at[pl.ds(r0, rows_per_unit)],
                        o_hbm.at[pl.ds(r0, rows_per_unit)])

        def body(i_vmem, o_vmem):
            pltpu.sync_copy(cb_hbm.at[i_vmem.at[0]], o_vmem)

        pltpu.emit_pipeline(
            body,
            grid=(n // GATHER_WINDOW,),
            in_specs=[pl.BlockSpec((1, GATHER_WINDOW),
                                   index_map=lambda i: (0, i))],
            out_specs=[pl.BlockSpec((GATHER_WINDOW, D),
                                    index_map=lambda i: (i + win0, 0))],
            core_axis_name=("core", "subcore"),
            dimension_semantics=(pltpu.PARALLEL,),
        )(i_hbm, o_hbm)

    return kern(codebook, idx2, part0)


CHUNK_TILES = (18, 18)


def kernel(uncompressed, mask, codebook):
    x_flat = uncompressed.reshape(T, D)
    c2, cb2 = _codebook_prep(codebook)
    parts = []
    tile0 = 0
    for tiles in CHUNK_TILES:
        idx_c = _nearest_idx(x_flat, cb2, c2, tile0, tiles)
        parts.append(_sc_gather(codebook, idx_c))
        tile0 += tiles
    compressed = jnp.concatenate(parts, axis=0).reshape(B, N, D)
    return (compressed, uncompressed, mask, codebook)
